# baseline (speedup 1.0000x reference)
import functools

import jax
import jax.numpy as jnp
from jax import lax
from jax.experimental import pallas as pl
from jax.experimental.pallas import tpu as pltpu
from jax.experimental.pallas import tpu_sc as plsc

VOCAB = 100000
EMBED_DIM = 128
CONTEXT = 20
NUM_NEURONS = 128

V_BLK = 4096
N_BLOCKS = -(-VOCAB // V_BLK)
V_PAD = N_BLOCKS * V_BLK


@functools.partial(
    pl.kernel,
    out_type=jax.ShapeDtypeStruct((CONTEXT, EMBED_DIM), jnp.float32),
    mesh=plsc.VectorSubcoreMesh(core_axis_name="c", subcore_axis_name="s"),
    scratch_types=[
        pltpu.VMEM((CONTEXT,), jnp.int32),
        pltpu.VMEM((CONTEXT, EMBED_DIM), jnp.float32),
        pltpu.SemaphoreType.DMA,
    ],
)
def _sc_gather(idx_hbm, table_hbm, out_hbm, idx_v, rows_v, sem):
    wid = lax.axis_index("s") * 2 + lax.axis_index("c")

    @pl.when(wid == 0)
    def _():
        pltpu.sync_copy(idx_hbm, idx_v)
        pltpu.async_copy(table_hbm.at[idx_v], rows_v, sem).wait()
        pltpu.sync_copy(rows_v, out_hbm)


def _tc_body(e_ref, w1_ref, b1_ref, w2_ref, b2_ref, out_ref, h_ref, ms_ref):
    i = pl.program_id(0)

    @pl.when(i == 0)
    def _init():
        h = lax.dot_general(
            e_ref[...], w1_ref[...],
            (((1,), (1,)), ((), ())),
            preferred_element_type=jnp.float32,
        )
        h_ref[...] = jnp.maximum(h + b1_ref[...], 0.0)
        ms_ref[0] = -1e30
        ms_ref[1] = 0.0

    blk = lax.dot_general(
        h_ref[...].astype(jnp.bfloat16), w2_ref[...].astype(jnp.bfloat16),
        (((1,), (1,)), ((), ())),
        preferred_element_type=jnp.float32,
    ) + b2_ref[...]

    out_ref[pl.ds(i, 1), :] = blk

    col = i * V_BLK + lax.broadcasted_iota(jnp.int32, (1, V_BLK), 1)
    blk_m = jnp.where(col < VOCAB, blk, -1e30)

    m_old = ms_ref[0]
    s_old = ms_ref[1]
    bm = jnp.max(blk_m)
    m_new = jnp.maximum(m_old, bm)
    s_new = s_old * jnp.exp(m_old - m_new) + jnp.sum(jnp.exp(blk_m - m_new))
    ms_ref[0] = m_new
    ms_ref[1] = s_new

    @pl.when(i == pl.num_programs(0) - 1)
    def _fin():
        lse = ms_ref[0] + jnp.log(ms_ref[1])
        out_ref[...] = out_ref[...] - lse


def kernel(inputs, emb, W1, b1, W2, b2):
    rows = _sc_gather(inputs, emb)
    e = rows.reshape(1, CONTEXT * EMBED_DIM)
    b1r = b1.reshape(1, NUM_NEURONS)
    b2p = jnp.pad(b2, (0, V_PAD - VOCAB)).reshape(1, V_PAD)

    res = pl.pallas_call(
        _tc_body,
        grid=(N_BLOCKS,),
        in_specs=[
            pl.BlockSpec((1, CONTEXT * EMBED_DIM), lambda i: (0, 0)),
            pl.BlockSpec((NUM_NEURONS, CONTEXT * EMBED_DIM), lambda i: (0, 0)),
            pl.BlockSpec((1, NUM_NEURONS), lambda i: (0, 0)),
            pl.BlockSpec((V_BLK, EMBED_DIM), lambda i: (i, 0)),
            pl.BlockSpec((1, V_BLK), lambda i: (0, i)),
        ],
        out_specs=pl.BlockSpec((N_BLOCKS, V_BLK), lambda i: (0, 0)),
        out_shape=jax.ShapeDtypeStruct((N_BLOCKS, V_BLK), jnp.float32),
        scratch_shapes=[
            pltpu.VMEM((1, NUM_NEURONS), jnp.float32),
            pltpu.SMEM((2,), jnp.float32),
        ],
    )(e, W1, b1r, W2, b2p)

    return res.reshape(1, V_PAD)[:, :VOCAB]


# speedup vs baseline: 1.1746x; 1.1746x over previous
import functools

import jax
import jax.numpy as jnp
from jax import lax
from jax.experimental import pallas as pl
from jax.experimental.pallas import tpu as pltpu
from jax.experimental.pallas import tpu_sc as plsc

VOCAB = 100000
EMBED_DIM = 128
CONTEXT = 20
NUM_NEURONS = 128

V_BLK = 8192
N_BLOCKS = -(-VOCAB // V_BLK)
V_PAD = N_BLOCKS * V_BLK


@functools.partial(
    pl.kernel,
    out_type=jax.ShapeDtypeStruct((CONTEXT, EMBED_DIM), jnp.float32),
    mesh=plsc.VectorSubcoreMesh(core_axis_name="c", subcore_axis_name="s"),
    scratch_types=[
        pltpu.VMEM((CONTEXT,), jnp.int32),
        pltpu.VMEM((CONTEXT, EMBED_DIM), jnp.float32),
        pltpu.SemaphoreType.DMA,
    ],
)
def _sc_gather(idx_hbm, table_hbm, out_hbm, idx_v, rows_v, sem):
    wid = lax.axis_index("s") * 2 + lax.axis_index("c")

    @pl.when(wid == 0)
    def _():
        pltpu.sync_copy(idx_hbm, idx_v)
        pltpu.async_copy(table_hbm.at[idx_v], rows_v, sem).wait()
        pltpu.sync_copy(rows_v, out_hbm)


def _tc_body(e_ref, w1_ref, b1_ref, w2_ref, b2_ref, out_ref, h_ref, ms_ref):
    i = pl.program_id(0)

    @pl.when(i == 0)
    def _init():
        h = lax.dot_general(
            e_ref[...], w1_ref[...],
            (((1,), (1,)), ((), ())),
            preferred_element_type=jnp.float32,
        )
        h_ref[...] = jnp.maximum(h + b1_ref[...], 0.0)
        ms_ref[0] = -1e30
        ms_ref[1] = 0.0

    blk = lax.dot_general(
        h_ref[...].astype(jnp.bfloat16), w2_ref[...].astype(jnp.bfloat16),
        (((1,), (1,)), ((), ())),
        preferred_element_type=jnp.float32,
    ) + b2_ref[...]

    out_ref[pl.ds(i, 1), :] = blk

    col = i * V_BLK + lax.broadcasted_iota(jnp.int32, (1, V_BLK), 1)
    blk_m = jnp.where(col < VOCAB, blk, -1e30)

    m_old = ms_ref[0]
    s_old = ms_ref[1]
    bm = jnp.max(blk_m)
    m_new = jnp.maximum(m_old, bm)
    s_new = s_old * jnp.exp(m_old - m_new) + jnp.sum(jnp.exp(blk_m - m_new))
    ms_ref[0] = m_new
    ms_ref[1] = s_new

    @pl.when(i == pl.num_programs(0) - 1)
    def _fin():
        lse = ms_ref[0] + jnp.log(ms_ref[1])
        out_ref[...] = out_ref[...] - lse


def kernel(inputs, emb, W1, b1, W2, b2):
    rows = _sc_gather(inputs, emb)
    e = rows.reshape(1, CONTEXT * EMBED_DIM)
    b1r = b1.reshape(1, NUM_NEURONS)
    b2p = jnp.pad(b2, (0, V_PAD - VOCAB)).reshape(1, V_PAD)

    res = pl.pallas_call(
        _tc_body,
        grid=(N_BLOCKS,),
        in_specs=[
            pl.BlockSpec((1, CONTEXT * EMBED_DIM), lambda i: (0, 0)),
            pl.BlockSpec((NUM_NEURONS, CONTEXT * EMBED_DIM), lambda i: (0, 0)),
            pl.BlockSpec((1, NUM_NEURONS), lambda i: (0, 0)),
            pl.BlockSpec((V_BLK, EMBED_DIM), lambda i: (i, 0)),
            pl.BlockSpec((1, V_BLK), lambda i: (0, i)),
        ],
        out_specs=pl.BlockSpec((N_BLOCKS, V_BLK), lambda i: (0, 0)),
        out_shape=jax.ShapeDtypeStruct((N_BLOCKS, V_BLK), jnp.float32),
        scratch_shapes=[
            pltpu.VMEM((1, NUM_NEURONS), jnp.float32),
            pltpu.SMEM((2,), jnp.float32),
        ],
    )(e, W1, b1r, W2, b2p)

    return res.reshape(1, V_PAD)[:, :VOCAB]


# speedup vs baseline: 1.2643x; 1.0763x over previous
import functools

import jax
import jax.numpy as jnp
from jax import lax
from jax.experimental import pallas as pl
from jax.experimental.pallas import tpu as pltpu
from jax.experimental.pallas import tpu_sc as plsc

VOCAB = 100000
EMBED_DIM = 128
CONTEXT = 20
NUM_NEURONS = 128

V_BLK = 12800
N_BLOCKS = -(-VOCAB // V_BLK)
V_PAD = N_BLOCKS * V_BLK


@functools.partial(
    pl.kernel,
    out_type=jax.ShapeDtypeStruct((CONTEXT, EMBED_DIM), jnp.float32),
    mesh=plsc.VectorSubcoreMesh(core_axis_name="c", subcore_axis_name="s"),
    scratch_types=[
        pltpu.VMEM((CONTEXT,), jnp.int32),
        pltpu.VMEM((CONTEXT, EMBED_DIM), jnp.float32),
        pltpu.SemaphoreType.DMA,
    ],
)
def _sc_gather(idx_hbm, table_hbm, out_hbm, idx_v, rows_v, sem):
    wid = lax.axis_index("s") * 2 + lax.axis_index("c")

    @pl.when(wid == 0)
    def _():
        pltpu.sync_copy(idx_hbm, idx_v)
        pltpu.async_copy(table_hbm.at[idx_v], rows_v, sem).wait()
        pltpu.sync_copy(rows_v, out_hbm)


def _tc_body(e_ref, w1_ref, b1_ref, w2_ref, b2_ref, out_ref, h_ref, ms_ref):
    i = pl.program_id(0)

    @pl.when(i == 0)
    def _init():
        h = lax.dot_general(
            e_ref[...], w1_ref[...],
            (((1,), (1,)), ((), ())),
            preferred_element_type=jnp.float32,
        )
        h_ref[...] = jnp.maximum(h + b1_ref[...], 0.0)
        ms_ref[0] = -1e30
        ms_ref[1] = 0.0

    blk = lax.dot_general(
        h_ref[...].astype(jnp.bfloat16), w2_ref[...].astype(jnp.bfloat16),
        (((1,), (1,)), ((), ())),
        preferred_element_type=jnp.float32,
    ) + b2_ref[...]

    out_ref[pl.ds(i, 1), :] = blk

    col = i * V_BLK + lax.broadcasted_iota(jnp.int32, (1, V_BLK), 1)
    blk_m = jnp.where(col < VOCAB, blk, -1e30)

    m_old = ms_ref[0]
    s_old = ms_ref[1]
    bm = jnp.max(blk_m)
    m_new = jnp.maximum(m_old, bm)
    s_new = s_old * jnp.exp(m_old - m_new) + jnp.sum(jnp.exp(blk_m - m_new))
    ms_ref[0] = m_new
    ms_ref[1] = s_new

    @pl.when(i == pl.num_programs(0) - 1)
    def _fin():
        lse = ms_ref[0] + jnp.log(ms_ref[1])
        out_ref[...] = out_ref[...] - lse


def kernel(inputs, emb, W1, b1, W2, b2):
    rows = _sc_gather(inputs, emb)
    e = rows.reshape(1, CONTEXT * EMBED_DIM)
    b1r = b1.reshape(1, NUM_NEURONS)
    b2p = jnp.pad(b2, (0, V_PAD - VOCAB)).reshape(1, V_PAD)

    res = pl.pallas_call(
        _tc_body,
        grid=(N_BLOCKS,),
        in_specs=[
            pl.BlockSpec((1, CONTEXT * EMBED_DIM), lambda i: (0, 0)),
            pl.BlockSpec((NUM_NEURONS, CONTEXT * EMBED_DIM), lambda i: (0, 0)),
            pl.BlockSpec((1, NUM_NEURONS), lambda i: (0, 0)),
            pl.BlockSpec((V_BLK, EMBED_DIM), lambda i: (i, 0)),
            pl.BlockSpec((1, V_BLK), lambda i: (0, i)),
        ],
        out_specs=pl.BlockSpec((N_BLOCKS, V_BLK), lambda i: (0, 0)),
        out_shape=jax.ShapeDtypeStruct((N_BLOCKS, V_BLK), jnp.float32),
        scratch_shapes=[
            pltpu.VMEM((1, NUM_NEURONS), jnp.float32),
            pltpu.SMEM((2,), jnp.float32),
        ],
    )(e, W1, b1r, W2, b2p)

    return res.reshape(1, V_PAD)[:, :VOCAB]


# speedup vs baseline: 1.2976x; 1.0264x over previous
import functools

import jax
import jax.numpy as jnp
from jax import lax
from jax.experimental import pallas as pl
from jax.experimental.pallas import tpu as pltpu
from jax.experimental.pallas import tpu_sc as plsc

VOCAB = 100000
EMBED_DIM = 128
CONTEXT = 20
NUM_NEURONS = 128

V_BLK = 25600
N_BLOCKS = -(-VOCAB // V_BLK)
V_PAD = N_BLOCKS * V_BLK


@functools.partial(
    pl.kernel,
    out_type=jax.ShapeDtypeStruct((CONTEXT, EMBED_DIM), jnp.float32),
    mesh=plsc.VectorSubcoreMesh(core_axis_name="c", subcore_axis_name="s"),
    scratch_types=[
        pltpu.VMEM((CONTEXT,), jnp.int32),
        pltpu.VMEM((CONTEXT, EMBED_DIM), jnp.float32),
        pltpu.SemaphoreType.DMA,
    ],
)
def _sc_gather(idx_hbm, table_hbm, out_hbm, idx_v, rows_v, sem):
    wid = lax.axis_index("s") * 2 + lax.axis_index("c")

    @pl.when(wid == 0)
    def _():
        pltpu.sync_copy(idx_hbm, idx_v)
        pltpu.async_copy(table_hbm.at[idx_v], rows_v, sem).wait()
        pltpu.sync_copy(rows_v, out_hbm)


def _tc_body(e_ref, w1_ref, b1_ref, w2_ref, b2_ref, out_ref, h_ref, ms_ref):
    i = pl.program_id(0)

    @pl.when(i == 0)
    def _init():
        h = lax.dot_general(
            e_ref[...], w1_ref[...],
            (((1,), (1,)), ((), ())),
            preferred_element_type=jnp.float32,
        )
        h_ref[...] = jnp.maximum(h + b1_ref[...], 0.0)
        ms_ref[0] = -1e30
        ms_ref[1] = 0.0

    blk = lax.dot_general(
        h_ref[...].astype(jnp.bfloat16), w2_ref[...].astype(jnp.bfloat16),
        (((1,), (1,)), ((), ())),
        preferred_element_type=jnp.float32,
    ) + b2_ref[...]

    out_ref[pl.ds(i, 1), :] = blk

    col = i * V_BLK + lax.broadcasted_iota(jnp.int32, (1, V_BLK), 1)
    blk_m = jnp.where(col < VOCAB, blk, -1e30)

    m_old = ms_ref[0]
    s_old = ms_ref[1]
    bm = jnp.max(blk_m)
    m_new = jnp.maximum(m_old, bm)
    s_new = s_old * jnp.exp(m_old - m_new) + jnp.sum(jnp.exp(blk_m - m_new))
    ms_ref[0] = m_new
    ms_ref[1] = s_new

    @pl.when(i == pl.num_programs(0) - 1)
    def _fin():
        lse = ms_ref[0] + jnp.log(ms_ref[1])
        out_ref[...] = out_ref[...] - lse


def kernel(inputs, emb, W1, b1, W2, b2):
    rows = _sc_gather(inputs, emb)
    e = rows.reshape(1, CONTEXT * EMBED_DIM)
    b1r = b1.reshape(1, NUM_NEURONS)
    b2p = jnp.pad(b2, (0, V_PAD - VOCAB)).reshape(1, V_PAD)

    res = pl.pallas_call(
        _tc_body,
        grid=(N_BLOCKS,),
        in_specs=[
            pl.BlockSpec((1, CONTEXT * EMBED_DIM), lambda i: (0, 0)),
            pl.BlockSpec((NUM_NEURONS, CONTEXT * EMBED_DIM), lambda i: (0, 0)),
            pl.BlockSpec((1, NUM_NEURONS), lambda i: (0, 0)),
            pl.BlockSpec((V_BLK, EMBED_DIM), lambda i: (i, 0)),
            pl.BlockSpec((1, V_BLK), lambda i: (0, i)),
        ],
        out_specs=pl.BlockSpec((N_BLOCKS, V_BLK), lambda i: (0, 0)),
        out_shape=jax.ShapeDtypeStruct((N_BLOCKS, V_BLK), jnp.float32),
        scratch_shapes=[
            pltpu.VMEM((1, NUM_NEURONS), jnp.float32),
            pltpu.SMEM((2,), jnp.float32),
        ],
    )(e, W1, b1r, W2, b2p)

    return res.reshape(1, V_PAD)[:, :VOCAB]
